# baseline (speedup 1.0000x reference)
import functools

import jax
import jax.numpy as jnp
from jax import lax
from jax.experimental import pallas as pl
from jax.experimental.pallas import tpu as pltpu
from jax.experimental.pallas import tpu_sc as plsc

N_TOKENS = 8192
N_EXPERTS = 8
N_CORES = 2
N_SUBCORES = 16
LANES = 16


def _sc_histogram(idx2d, wflat):
    total_pairs = idx2d.shape[0] * idx2d.shape[1]
    pairs_per_tile = total_pairs // (N_CORES * N_SUBCORES)
    rows_per_tile = pairs_per_tile // idx2d.shape[1]
    chunk = idx2d.shape[1]
    tok_per_tile = N_TOKENS // N_SUBCORES

    mesh = plsc.VectorSubcoreMesh(core_axis_name="c", subcore_axis_name="s")

    @functools.partial(
        pl.kernel,
        mesh=mesh,
        compiler_params=pltpu.CompilerParams(needs_layout_passes=False),
        out_type=jax.ShapeDtypeStruct((N_CORES, 2, N_TOKENS), jnp.float32),
        scratch_types=[
            pltpu.VMEM((rows_per_tile, chunk), jnp.int32),
            pltpu.VMEM((wflat.shape[0],), jnp.float32),
            pltpu.VMEM((rows_per_tile, chunk), jnp.float32),
            pltpu.VMEM((rows_per_tile, chunk), jnp.float32),
            pltpu.VMEM((tok_per_tile,), jnp.float32),
            pltpu.VMEM_SHARED((N_TOKENS,), jnp.float32),
            pltpu.VMEM_SHARED((N_TOKENS,), jnp.float32),
        ],
    )
    def hist_kernel(idx_hbm, w_hbm, out_hbm,
                    idx_v, wtab_v, vals_v, ones_v, zero_v, accw_sh, accc_sh):
        cid = lax.axis_index("c")
        sid = lax.axis_index("s")
        row0 = (cid * N_SUBCORES + sid) * rows_per_tile

        pltpu.sync_copy(idx_hbm.at[pl.ds(row0, rows_per_tile)], idx_v)
        pltpu.sync_copy(w_hbm, wtab_v)

        zeros16 = jnp.zeros((LANES,), jnp.float32)
        ones16 = jnp.ones((LANES,), jnp.float32)
        for i in range(tok_per_tile // LANES):
            zero_v[pl.ds(i * LANES, LANES)] = zeros16
        e_vec = jnp.bitwise_and(lax.iota(jnp.int32, LANES), N_EXPERTS - 1)
        for j in range(rows_per_tile):
            for k in range(chunk // LANES):
                tok = idx_v[j, pl.ds(k * LANES, LANES)]
                fidx = tok * N_EXPERTS + e_vec
                w = plsc.load_gather(wtab_v, [fidx])
                vals_v[j, pl.ds(k * LANES, LANES)] = w
                ones_v[j, pl.ds(k * LANES, LANES)] = ones16

        pltpu.sync_copy(zero_v, accw_sh.at[pl.ds(sid * tok_per_tile, tok_per_tile)])
        pltpu.sync_copy(zero_v, accc_sh.at[pl.ds(sid * tok_per_tile, tok_per_tile)])
        plsc.subcore_barrier()

        for j in range(rows_per_tile):
            pltpu.sync_copy(vals_v.at[j], accw_sh.at[idx_v.at[j]], add=True)
            pltpu.sync_copy(ones_v.at[j], accc_sh.at[idx_v.at[j]], add=True)
        plsc.subcore_barrier()

        @pl.when(sid == 0)
        def _():
            pltpu.sync_copy(accw_sh, out_hbm.at[cid, 0])
            pltpu.sync_copy(accc_sh, out_hbm.at[cid, 1])

    return hist_kernel(idx2d, wflat)


def _ffn_body(ratio_scale, x_ref, w1_ref, b1_ref, w2_ref, b2_ref, parts_ref,
              out_ref, ratio_ref):
    x = x_ref[...]
    h = jnp.dot(x, w1_ref[...], preferred_element_type=jnp.float32) + b1_ref[...]
    h = jnp.maximum(h, 0.0)
    nnz = jnp.sum((h > 0.0).astype(jnp.float32), axis=1)
    out = jnp.dot(h.astype(jnp.bfloat16), w2_ref[...],
                  preferred_element_type=jnp.float32) + b2_ref[...]
    parts = parts_ref[...]
    tw = parts[0, 0, :] + parts[1, 0, :]
    cnt = parts[0, 1, :] + parts[1, 1, :]
    out_ref[...] = out * tw[:, None]

    @pl.when(pl.program_id(0) == 0)
    def _():
        ratio_ref[0, 0] = 0.0

    ratio_ref[0, 0] += jnp.sum(nnz * cnt) * ratio_scale


def kernel(inputs, inputs_weight, top_idx, W1, b1, W2, b2):
    n_tok, d_model = inputs.shape
    d_ff = W1.shape[1]
    cap, n_exp = top_idx.shape

    idx2d = top_idx.astype(jnp.int32).reshape(128, 128)
    wflat = inputs_weight.reshape(-1)
    parts = _sc_histogram(idx2d, wflat)

    blk_m = 512
    grid = (n_tok // blk_m,)
    ratio_scale = 1.0 / (n_exp * cap * d_ff)

    out, ratio = pl.pallas_call(
        functools.partial(_ffn_body, ratio_scale),
        grid=grid,
        in_specs=[
            pl.BlockSpec((blk_m, d_model), lambda i: (i, 0)),
            pl.BlockSpec((d_model, d_ff), lambda i: (0, 0)),
            pl.BlockSpec((1, d_ff), lambda i: (0, 0)),
            pl.BlockSpec((d_ff, d_model), lambda i: (0, 0)),
            pl.BlockSpec((1, d_model), lambda i: (0, 0)),
            pl.BlockSpec((2, 2, blk_m), lambda i: (0, 0, i)),
        ],
        out_specs=[
            pl.BlockSpec((blk_m, d_model), lambda i: (i, 0)),
            pl.BlockSpec((1, 1), lambda i: (0, 0), memory_space=pltpu.SMEM),
        ],
        out_shape=[
            jax.ShapeDtypeStruct((n_tok, d_model), jnp.float32),
            jax.ShapeDtypeStruct((1, 1), jnp.float32),
        ],
    )(inputs.astype(jnp.bfloat16), W1.astype(jnp.bfloat16),
      b1.reshape(1, d_ff), W2.astype(jnp.bfloat16),
      b2.reshape(1, d_model), parts)
    return out, ratio[0, 0]


# speedup vs baseline: 1.0875x; 1.0875x over previous
import functools

import jax
import jax.numpy as jnp
from jax import lax
from jax.experimental import pallas as pl
from jax.experimental.pallas import tpu as pltpu
from jax.experimental.pallas import tpu_sc as plsc

N_TOKENS = 8192
N_EXPERTS = 8
N_CORES = 2
N_SUBCORES = 16
LANES = 16


def _sc_histogram(idx2d, wflat):
    total_pairs = idx2d.shape[0] * idx2d.shape[1]
    pairs_per_tile = total_pairs // (N_CORES * N_SUBCORES)
    rows_per_tile = pairs_per_tile // idx2d.shape[1]
    chunk = idx2d.shape[1]
    tok_per_tile = N_TOKENS // N_SUBCORES

    mesh = plsc.VectorSubcoreMesh(core_axis_name="c", subcore_axis_name="s")

    @functools.partial(
        pl.kernel,
        mesh=mesh,
        compiler_params=pltpu.CompilerParams(needs_layout_passes=False),
        out_type=jax.ShapeDtypeStruct((N_CORES, 2, N_TOKENS), jnp.float32),
        scratch_types=[
            pltpu.VMEM((rows_per_tile, chunk), jnp.int32),
            pltpu.VMEM((wflat.shape[0],), jnp.float32),
            pltpu.VMEM((rows_per_tile, chunk), jnp.float32),
            pltpu.VMEM((rows_per_tile, chunk), jnp.float32),
            pltpu.VMEM((tok_per_tile,), jnp.float32),
            pltpu.VMEM_SHARED((N_TOKENS,), jnp.float32),
            pltpu.VMEM_SHARED((N_TOKENS,), jnp.float32),
        ],
    )
    def hist_kernel(idx_hbm, w_hbm, out_hbm,
                    idx_v, wtab_v, vals_v, ones_v, zero_v, accw_sh, accc_sh):
        cid = lax.axis_index("c")
        sid = lax.axis_index("s")
        row0 = (cid * N_SUBCORES + sid) * rows_per_tile

        pltpu.sync_copy(idx_hbm.at[pl.ds(row0, rows_per_tile)], idx_v)
        pltpu.sync_copy(w_hbm, wtab_v)

        zeros16 = jnp.zeros((LANES,), jnp.float32)
        ones16 = jnp.ones((LANES,), jnp.float32)
        for i in range(tok_per_tile // LANES):
            zero_v[pl.ds(i * LANES, LANES)] = zeros16
        e_vec = jnp.bitwise_and(lax.iota(jnp.int32, LANES), N_EXPERTS - 1)
        for j in range(rows_per_tile):
            for k in range(chunk // LANES):
                tok = idx_v[j, pl.ds(k * LANES, LANES)]
                fidx = tok * N_EXPERTS + e_vec
                w = plsc.load_gather(wtab_v, [fidx])
                vals_v[j, pl.ds(k * LANES, LANES)] = w
                ones_v[j, pl.ds(k * LANES, LANES)] = ones16

        pltpu.sync_copy(zero_v, accw_sh.at[pl.ds(sid * tok_per_tile, tok_per_tile)])
        pltpu.sync_copy(zero_v, accc_sh.at[pl.ds(sid * tok_per_tile, tok_per_tile)])
        plsc.subcore_barrier()

        for j in range(rows_per_tile):
            pltpu.sync_copy(vals_v.at[j], accw_sh.at[idx_v.at[j]], add=True)
            pltpu.sync_copy(ones_v.at[j], accc_sh.at[idx_v.at[j]], add=True)
        plsc.subcore_barrier()

        @pl.when(sid == 0)
        def _():
            pltpu.sync_copy(accw_sh, out_hbm.at[cid, 0])
            pltpu.sync_copy(accc_sh, out_hbm.at[cid, 1])

    return hist_kernel(idx2d, wflat)


def _ffn_body(ratio_scale, x_ref, w1_ref, b1_ref, w2_ref, b2_ref, parts_ref,
              out_ref, ratio_ref):
    x = x_ref[...].astype(jnp.bfloat16)
    h = jnp.dot(x, w1_ref[...], preferred_element_type=jnp.float32) + b1_ref[...]
    h = jnp.maximum(h, 0.0)
    nnz = jnp.sum((h > 0.0).astype(jnp.float32), axis=1)
    out = jnp.dot(h.astype(jnp.bfloat16), w2_ref[...],
                  preferred_element_type=jnp.float32) + b2_ref[...]
    parts = parts_ref[...]
    tw = parts[0, 0, :] + parts[1, 0, :]
    cnt = parts[0, 1, :] + parts[1, 1, :]
    out_ref[...] = out * tw[:, None]

    @pl.when(pl.program_id(0) == 0)
    def _():
        ratio_ref[0, 0] = 0.0

    ratio_ref[0, 0] += jnp.sum(nnz * cnt) * ratio_scale


def kernel(inputs, inputs_weight, top_idx, W1, b1, W2, b2):
    n_tok, d_model = inputs.shape
    d_ff = W1.shape[1]
    cap, n_exp = top_idx.shape

    idx2d = top_idx.astype(jnp.int32).reshape(128, 128)
    wflat = inputs_weight.reshape(-1)
    parts = _sc_histogram(idx2d, wflat)

    blk_m = 512
    grid = (n_tok // blk_m,)
    ratio_scale = 1.0 / (n_exp * cap * d_ff)

    out, ratio = pl.pallas_call(
        functools.partial(_ffn_body, ratio_scale),
        grid=grid,
        in_specs=[
            pl.BlockSpec((blk_m, d_model), lambda i: (i, 0)),
            pl.BlockSpec((d_model, d_ff), lambda i: (0, 0)),
            pl.BlockSpec((1, d_ff), lambda i: (0, 0)),
            pl.BlockSpec((d_ff, d_model), lambda i: (0, 0)),
            pl.BlockSpec((1, d_model), lambda i: (0, 0)),
            pl.BlockSpec((2, 2, blk_m), lambda i: (0, 0, i)),
        ],
        out_specs=[
            pl.BlockSpec((blk_m, d_model), lambda i: (i, 0)),
            pl.BlockSpec((1, 1), lambda i: (0, 0), memory_space=pltpu.SMEM),
        ],
        out_shape=[
            jax.ShapeDtypeStruct((n_tok, d_model), jnp.float32),
            jax.ShapeDtypeStruct((1, 1), jnp.float32),
        ],
    )(inputs, W1.astype(jnp.bfloat16),
      b1.reshape(1, d_ff), W2.astype(jnp.bfloat16),
      b2.reshape(1, d_model), parts)
    return out, ratio[0, 0]


# speedup vs baseline: 1.1709x; 1.0767x over previous
import functools

import jax
import jax.numpy as jnp
from jax import lax
from jax.experimental import pallas as pl
from jax.experimental.pallas import tpu as pltpu
from jax.experimental.pallas import tpu_sc as plsc

N_TOKENS = 8192
N_EXPERTS = 8
N_CORES = 2
N_SUBCORES = 16
LANES = 16


def _sc_histogram(idx2d):
    n_bins = N_TOKENS * N_EXPERTS
    total_pairs = idx2d.shape[0] * idx2d.shape[1]
    pairs_per_tile = total_pairs // (N_CORES * N_SUBCORES)
    rows_per_tile = pairs_per_tile // idx2d.shape[1]
    chunk = idx2d.shape[1]
    bins_per_tile = n_bins // N_SUBCORES

    mesh = plsc.VectorSubcoreMesh(core_axis_name="c", subcore_axis_name="s")

    @functools.partial(
        pl.kernel,
        mesh=mesh,
        compiler_params=pltpu.CompilerParams(needs_layout_passes=False),
        out_type=jax.ShapeDtypeStruct((N_CORES, n_bins), jnp.float32),
        scratch_types=[
            pltpu.VMEM((rows_per_tile, chunk), jnp.int32),
            pltpu.VMEM((rows_per_tile, chunk), jnp.int32),
            pltpu.VMEM((rows_per_tile, chunk), jnp.float32),
            pltpu.VMEM((bins_per_tile,), jnp.float32),
            pltpu.VMEM_SHARED((n_bins,), jnp.float32),
        ],
    )
    def hist_kernel(idx_hbm, out_hbm, idx_v, fidx_v, ones_v, zero_v, acc_sh):
        cid = lax.axis_index("c")
        sid = lax.axis_index("s")
        row0 = (cid * N_SUBCORES + sid) * rows_per_tile

        pltpu.sync_copy(idx_hbm.at[pl.ds(row0, rows_per_tile)], idx_v)

        zeros16 = jnp.zeros((LANES,), jnp.float32)
        ones16 = jnp.ones((LANES,), jnp.float32)
        for i in range(bins_per_tile // LANES):
            zero_v[pl.ds(i * LANES, LANES)] = zeros16
        e_vec = jnp.bitwise_and(lax.iota(jnp.int32, LANES), N_EXPERTS - 1)
        for j in range(rows_per_tile):
            for k in range(chunk // LANES):
                tok = idx_v[j, pl.ds(k * LANES, LANES)]
                fidx_v[j, pl.ds(k * LANES, LANES)] = tok * N_EXPERTS + e_vec
                ones_v[j, pl.ds(k * LANES, LANES)] = ones16

        pltpu.sync_copy(zero_v, acc_sh.at[pl.ds(sid * bins_per_tile, bins_per_tile)])
        plsc.subcore_barrier()

        for j in range(rows_per_tile):
            pltpu.sync_copy(ones_v.at[j], acc_sh.at[fidx_v.at[j]], add=True)
        plsc.subcore_barrier()

        @pl.when(sid == 0)
        def _():
            pltpu.sync_copy(acc_sh, out_hbm.at[cid])

    return hist_kernel(idx2d)


def _ffn_body(ratio_scale, x_ref, w1_ref, b1_ref, w2_ref, b2_ref, cnt_ref,
              iw_ref, out_ref, ratio_ref):
    x = x_ref[...]
    h = jnp.dot(x, w1_ref[...], preferred_element_type=jnp.float32) + b1_ref[...]
    h = jnp.maximum(h, 0.0)
    nnz = jnp.sum((h > 0.0).astype(jnp.float32), axis=1)
    out = jnp.dot(h, w2_ref[...], preferred_element_type=jnp.float32) + b2_ref[...]
    cnt = cnt_ref[0] + cnt_ref[1]
    tw = jnp.sum(cnt * iw_ref[...], axis=1)
    ctot = jnp.sum(cnt, axis=1)
    out_ref[...] = out * tw[:, None]

    @pl.when(pl.program_id(0) == 0)
    def _():
        ratio_ref[0, 0] = 0.0

    ratio_ref[0, 0] += jnp.sum(nnz * ctot) * ratio_scale


def kernel(inputs, inputs_weight, top_idx, W1, b1, W2, b2):
    n_tok, d_model = inputs.shape
    d_ff = W1.shape[1]
    cap, n_exp = top_idx.shape

    idx2d = top_idx.astype(jnp.int32).reshape(128, 128)
    counts = _sc_histogram(idx2d).reshape(N_CORES, n_tok, n_exp)

    blk_m = 512
    grid = (n_tok // blk_m,)
    ratio_scale = 1.0 / (n_exp * cap * d_ff)

    out, ratio = pl.pallas_call(
        functools.partial(_ffn_body, ratio_scale),
        grid=grid,
        in_specs=[
            pl.BlockSpec((blk_m, d_model), lambda i: (i, 0)),
            pl.BlockSpec((d_model, d_ff), lambda i: (0, 0)),
            pl.BlockSpec((1, d_ff), lambda i: (0, 0)),
            pl.BlockSpec((d_ff, d_model), lambda i: (0, 0)),
            pl.BlockSpec((1, d_model), lambda i: (0, 0)),
            pl.BlockSpec((2, blk_m, n_exp), lambda i: (0, i, 0)),
            pl.BlockSpec((blk_m, n_exp), lambda i: (i, 0)),
        ],
        out_specs=[
            pl.BlockSpec((blk_m, d_model), lambda i: (i, 0)),
            pl.BlockSpec((1, 1), lambda i: (0, 0), memory_space=pltpu.SMEM),
        ],
        out_shape=[
            jax.ShapeDtypeStruct((n_tok, d_model), jnp.float32),
            jax.ShapeDtypeStruct((1, 1), jnp.float32),
        ],
    )(inputs, W1, b1.reshape(1, d_ff), W2, b2.reshape(1, d_model), counts,
      inputs_weight)
    return out, ratio[0, 0]
